# baseline (speedup 1.0000x reference)
import jax
import jax.numpy as jnp
from jax import lax
from jax.experimental import pallas as pl
from jax.experimental.pallas import tpu as pltpu

_TV = 4096
_NFULL = 24
_TAIL = 1696
_NS = 512
_NR = 256
_NSPLIT = 4
_SR = _NS // _NSPLIT


def _mm_body(z_ref, b_ref, w_hbm, o_ref, oh_ref, wbufs, wtail, sems, tsem):
    t = pl.program_id(0)

    def issue(blk, buf):
        for s in range(_NSPLIT):
            pltpu.make_async_copy(
                w_hbm.at[pl.ds(s * _SR, _SR), pl.ds(blk * _TV, _TV)],
                wbufs.at[buf, pl.ds(s * _SR, _SR)],
                sems.at[buf, s]).start()

    def wait(buf):
        for s in range(_NSPLIT):
            pltpu.make_async_copy(
                w_hbm.at[pl.ds(s * _SR, _SR), pl.ds(0, _TV)],
                wbufs.at[buf, pl.ds(s * _SR, _SR)],
                sems.at[buf, s]).wait()

    @pl.when(t == 0)
    def _prologue():
        states = lax.broadcasted_iota(jnp.int32, (_NR, _NS), 1)
        oh_ref[...] = (states == z_ref[...]).astype(jnp.bfloat16)
        pltpu.make_async_copy(
            w_hbm.at[:, pl.ds(_NFULL * _TV, _TAIL)], wtail, tsem).start()
        issue(0, 0)
        issue(1, 1)

    @pl.when((t >= 1) & (t + 1 < _NFULL))
    def _prefetch():
        issue(t + 1, (t + 1) % 2)

    @pl.when(t < _NFULL)
    def _full_step():
        wait(t % 2)
        acc = lax.dot_general(
            oh_ref[...], wbufs[t % 2].astype(jnp.bfloat16),
            (((1,), (0,)), ((), ())), preferred_element_type=jnp.float32)
        o_ref[...] = acc + b_ref[...]

    @pl.when(t == _NFULL)
    def _tail_step():
        pltpu.make_async_copy(
            w_hbm.at[:, pl.ds(_NFULL * _TV, _TAIL)], wtail, tsem).wait()
        acc = lax.dot_general(
            oh_ref[...], wtail[...].astype(jnp.bfloat16),
            (((1,), (0,)), ((), ())), preferred_element_type=jnp.float32)
        o_ref[:, : _TAIL] = acc + b_ref[:, : _TAIL]


def kernel(z, W, b):
    batch, seq = z.shape
    n = batch * seq
    num_states, vocab = W.shape
    zc = z.reshape(n, 1).astype(jnp.int32)
    b2 = b.reshape(1, vocab)

    out = pl.pallas_call(
        _mm_body,
        grid=(_NFULL + 1,),
        in_specs=[
            pl.BlockSpec((n, 1), lambda j: (0, 0)),
            pl.BlockSpec((1, _TV), lambda j: (0, j)),
            pl.BlockSpec(memory_space=pltpu.MemorySpace.HBM),
        ],
        out_specs=pl.BlockSpec((n, _TV), lambda j: (0, j)),
        scratch_shapes=[
            pltpu.VMEM((n, num_states), jnp.bfloat16),
            pltpu.VMEM((2, num_states, _TV), jnp.float32),
            pltpu.VMEM((num_states, _TAIL), jnp.float32),
            pltpu.SemaphoreType.DMA((2, _NSPLIT)),
            pltpu.SemaphoreType.DMA,
        ],
        out_shape=jax.ShapeDtypeStruct((n, vocab), jnp.float32),
    )(zc, b2, W)
    return out.reshape(batch, seq, vocab)
